# baseline (speedup 1.0000x reference)
import functools

import jax
import jax.numpy as jnp
from jax import lax
from jax.experimental import pallas as pl
from jax.experimental.pallas import tpu as pltpu
from jax.experimental.pallas import tpu_sc as plsc

EMB_DIM = 64
NUM_CORES = 2
NUM_SUBCORES = 16
NUM_WORKERS = NUM_CORES * NUM_SUBCORES
CHUNK = 128


def _make_lookup(total_rows: int):
    chunks_per_worker = total_rows // (NUM_WORKERS * CHUNK)
    mesh = plsc.VectorSubcoreMesh(core_axis_name="c", subcore_axis_name="s")

    @functools.partial(
        pl.kernel,
        mesh=mesh,
        out_type=jax.ShapeDtypeStruct((total_rows, EMB_DIM), jnp.float32),
        scratch_types=[
            pltpu.VMEM((chunks_per_worker, CHUNK), jnp.int32),
            pltpu.VMEM((CHUNK, EMB_DIM), jnp.float32),
            pltpu.SemaphoreType.DMA,
        ],
        compiler_params=pltpu.CompilerParams(use_tc_tiling_on_sc=False),
    )
    def lookup(idx_hbm, table_hbm, out_hbm, idx_v, rows_v, sem):
        wid = lax.axis_index("s") * NUM_CORES + lax.axis_index("c")
        pltpu.sync_copy(idx_hbm.at[wid], idx_v)
        base = wid * chunks_per_worker * CHUNK

        def step(j, carry):
            pltpu.async_copy(table_hbm.at[idx_v.at[j]], rows_v, sem).wait()
            pltpu.sync_copy(rows_v, out_hbm.at[pl.ds(base + j * CHUNK, CHUNK)])
            return carry

        lax.fori_loop(0, chunks_per_worker, step, 0)

    return lookup


def kernel(x, table):
    batch, hist = x.shape
    total = batch * hist
    chunks_per_worker = total // (NUM_WORKERS * CHUNK)
    idx3d = x.reshape(NUM_WORKERS, chunks_per_worker, CHUNK)
    out = _make_lookup(total)(idx3d, table)
    return out.reshape(batch, hist, EMB_DIM)


# speedup vs baseline: 1.0391x; 1.0391x over previous
import functools

import jax
import jax.numpy as jnp
from jax import lax
from jax.experimental import pallas as pl
from jax.experimental.pallas import tpu as pltpu
from jax.experimental.pallas import tpu_sc as plsc

EMB_DIM = 64
NUM_CORES = 2
NUM_SUBCORES = 16
NUM_WORKERS = NUM_CORES * NUM_SUBCORES
CHUNK = 640


def _make_lookup(total_rows: int):
    chunks_per_worker = total_rows // (NUM_WORKERS * CHUNK)
    mesh = plsc.VectorSubcoreMesh(core_axis_name="c", subcore_axis_name="s")

    @functools.partial(
        pl.kernel,
        mesh=mesh,
        out_type=jax.ShapeDtypeStruct((total_rows, EMB_DIM), jnp.float32),
        scratch_types=[
            pltpu.VMEM((chunks_per_worker, CHUNK), jnp.int32),
            pltpu.VMEM((CHUNK, EMB_DIM), jnp.float32),
            pltpu.SemaphoreType.DMA,
        ],
        compiler_params=pltpu.CompilerParams(use_tc_tiling_on_sc=False),
    )
    def lookup(idx_hbm, table_hbm, out_hbm, idx_v, rows_v, sem):
        wid = lax.axis_index("s") * NUM_CORES + lax.axis_index("c")
        pltpu.sync_copy(idx_hbm.at[wid], idx_v)
        base = wid * chunks_per_worker * CHUNK

        def step(j, carry):
            pltpu.async_copy(table_hbm.at[idx_v.at[j]], rows_v, sem).wait()
            pltpu.sync_copy(rows_v, out_hbm.at[pl.ds(base + j * CHUNK, CHUNK)])
            return carry

        lax.fori_loop(0, chunks_per_worker, step, 0)

    return lookup


def kernel(x, table):
    batch, hist = x.shape
    total = batch * hist
    chunks_per_worker = total // (NUM_WORKERS * CHUNK)
    idx3d = x.reshape(NUM_WORKERS, chunks_per_worker, CHUNK)
    out = _make_lookup(total)(idx3d, table)
    return out.reshape(batch, hist, EMB_DIM)


# speedup vs baseline: 1.0438x; 1.0045x over previous
import functools

import jax
import jax.numpy as jnp
from jax import lax
from jax.experimental import pallas as pl
from jax.experimental.pallas import tpu as pltpu
from jax.experimental.pallas import tpu_sc as plsc

EMB_DIM = 64
NUM_CORES = 2
NUM_SUBCORES = 16
NUM_WORKERS = NUM_CORES * NUM_SUBCORES
CHUNK = 640


def _make_lookup(total_rows: int):
    chunks_per_worker = total_rows // (NUM_WORKERS * CHUNK)
    mesh = plsc.VectorSubcoreMesh(core_axis_name="c", subcore_axis_name="s")

    @functools.partial(
        pl.kernel,
        mesh=mesh,
        out_type=jax.ShapeDtypeStruct((total_rows, EMB_DIM), jnp.float32),
        scratch_types=[
            pltpu.VMEM((chunks_per_worker, CHUNK), jnp.int32),
            pltpu.VMEM((CHUNK, EMB_DIM), jnp.float32),
            pltpu.VMEM((CHUNK, EMB_DIM), jnp.float32),
            pltpu.SemaphoreType.DMA,
            pltpu.SemaphoreType.DMA,
            pltpu.SemaphoreType.DMA,
            pltpu.SemaphoreType.DMA,
        ],
        compiler_params=pltpu.CompilerParams(use_tc_tiling_on_sc=False),
    )
    def lookup(idx_hbm, table_hbm, out_hbm, idx_v, buf0, buf1, sg0, sg1, sw0, sw1):
        wid = lax.axis_index("s") * NUM_CORES + lax.axis_index("c")
        pltpu.sync_copy(idx_hbm.at[wid], idx_v)
        base = wid * chunks_per_worker * CHUNK

        bufs = [buf0, buf1]
        sg = [sg0, sg1]
        sw = [sw0, sw1]
        gathers = [None, None]
        writebacks = [None, None]
        for j in range(chunks_per_worker):
            b = j % 2
            if writebacks[b] is not None:
                writebacks[b].wait()
            gathers[b] = pltpu.async_copy(table_hbm.at[idx_v.at[j]], bufs[b], sg[b])
            if j >= 1:
                pb = (j - 1) % 2
                gathers[pb].wait()
                writebacks[pb] = pltpu.async_copy(
                    bufs[pb], out_hbm.at[pl.ds(base + (j - 1) * CHUNK, CHUNK)], sw[pb]
                )
        last = (chunks_per_worker - 1) % 2
        gathers[last].wait()
        writebacks[last] = pltpu.async_copy(
            bufs[last],
            out_hbm.at[pl.ds(base + (chunks_per_worker - 1) * CHUNK, CHUNK)],
            sw[last],
        )
        writebacks[1 - last].wait()
        writebacks[last].wait()

    return lookup


def kernel(x, table):
    batch, hist = x.shape
    total = batch * hist
    chunks_per_worker = total // (NUM_WORKERS * CHUNK)
    idx3d = x.reshape(NUM_WORKERS, chunks_per_worker, CHUNK)
    out = _make_lookup(total)(idx3d, table)
    return out.reshape(batch, hist, EMB_DIM)


# speedup vs baseline: 1.0448x; 1.0010x over previous
import functools

import jax
import jax.numpy as jnp
from jax import lax
from jax.experimental import pallas as pl
from jax.experimental.pallas import tpu as pltpu
from jax.experimental.pallas import tpu_sc as plsc

EMB_DIM = 64
NUM_CORES = 2
NUM_SUBCORES = 16
NUM_WORKERS = NUM_CORES * NUM_SUBCORES
CHUNK = 128
STREAMS = 5
GROUP = CHUNK * STREAMS


def _make_lookup(total_rows: int):
    chunks_per_worker = total_rows // (NUM_WORKERS * CHUNK)
    groups_per_worker = chunks_per_worker // STREAMS
    mesh = plsc.VectorSubcoreMesh(core_axis_name="c", subcore_axis_name="s")

    @functools.partial(
        pl.kernel,
        mesh=mesh,
        out_type=jax.ShapeDtypeStruct((total_rows, EMB_DIM), jnp.float32),
        scratch_types=[
            pltpu.VMEM((chunks_per_worker, CHUNK), jnp.int32),
            pltpu.VMEM((GROUP, EMB_DIM), jnp.float32),
            pltpu.VMEM((GROUP, EMB_DIM), jnp.float32),
            pltpu.SemaphoreType.DMA,
            pltpu.SemaphoreType.DMA,
            pltpu.SemaphoreType.DMA,
            pltpu.SemaphoreType.DMA,
        ],
        compiler_params=pltpu.CompilerParams(use_tc_tiling_on_sc=False),
    )
    def lookup(idx_hbm, table_hbm, out_hbm, idx_v, buf0, buf1, sg0, sg1, sw0, sw1):
        wid = lax.axis_index("s") * NUM_CORES + lax.axis_index("c")
        pltpu.sync_copy(idx_hbm.at[wid], idx_v)
        base = wid * chunks_per_worker * CHUNK

        bufs = [buf0, buf1]
        sg = [sg0, sg1]
        sw = [sw0, sw1]
        gathers = [None, None]
        writebacks = [None, None]

        def fire_group(g, b):
            gathers[b] = [
                pltpu.async_copy(
                    table_hbm.at[idx_v.at[g * STREAMS + k]],
                    bufs[b].at[pl.ds(k * CHUNK, CHUNK)],
                    sg[b],
                )
                for k in range(STREAMS)
            ]

        def drain_and_writeback(g, b):
            for d in gathers[b]:
                d.wait()
            writebacks[b] = pltpu.async_copy(
                bufs[b], out_hbm.at[pl.ds(base + g * GROUP, GROUP)], sw[b]
            )

        for g in range(groups_per_worker):
            b = g % 2
            if writebacks[b] is not None:
                writebacks[b].wait()
            fire_group(g, b)
            if g >= 1:
                drain_and_writeback(g - 1, (g - 1) % 2)
        last_g = groups_per_worker - 1
        drain_and_writeback(last_g, last_g % 2)
        writebacks[0].wait()
        writebacks[1].wait()

    return lookup


def kernel(x, table):
    batch, hist = x.shape
    total = batch * hist
    chunks_per_worker = total // (NUM_WORKERS * CHUNK)
    idx3d = x.reshape(NUM_WORKERS, chunks_per_worker, CHUNK)
    out = _make_lookup(total)(idx3d, table)
    return out.reshape(batch, hist, EMB_DIM)
